# baseline (speedup 1.0000x reference)
import jax
import jax.numpy as jnp
import numpy as np
from jax.experimental import pallas as pl

K_NBR = 32
HEADS = 12


def _dist_qkv_kernel(xb_ref, xf_ref, wqkv_ref, mask_ref, q_ref, kv_ref):
    blk = pl.program_id(0)
    xb = xb_ref[...]
    xf = xf_ref[...]
    BR, _ = xb.shape
    S = xf.shape[0]
    g = jnp.dot(xb, xf.T, preferred_element_type=jnp.float32)
    x2b = jnp.sum(xb * xb, axis=1)
    x2f = jnp.sum(xf * xf, axis=1)
    d2 = x2b[:, None] + x2f[None, :] - 2.0 * g
    d2 = jnp.maximum(d2, 0.0)
    rows = jax.lax.broadcasted_iota(jnp.int32, (BR, S), 0) + blk * BR
    cols = jax.lax.broadcasted_iota(jnp.int32, (BR, S), 1)
    d2 = jnp.where(rows == cols, jnp.inf, d2)
    d = d2
    m = None
    for _ in range(K_NBR):
        m = jnp.min(d, axis=1, keepdims=True)
        d = jnp.where(d <= m, jnp.inf, d)
    mask_ref[...] = (d2 <= m).astype(jnp.int8)
    qkv = jnp.dot(xb, wqkv_ref[...], preferred_element_type=jnp.float32)
    D = xb.shape[1]
    scale = 1.0 / np.sqrt(D // HEADS)
    q_ref[...] = (qkv[:, :D] * scale).astype(jnp.bfloat16)
    kv_ref[...] = qkv[:, D:].astype(jnp.bfloat16)


def _attn_ffn_kernel(q_ref, kv_ref, mask_ref, x_ref, wo_ref, g1_ref,
                     b1n_ref, w1_ref, bf1_ref, w2_ref, bf2_ref, g2_ref,
                     b2n_ref, o_ref):
    D = q_ref.shape[1]
    dh = D // HEADS
    q = q_ref[...]
    kv = kv_ref[...]
    maskf = mask_ref[...].astype(jnp.float32)
    ctxs = []
    for h in range(HEADS):
        qh = q[:, h * dh:(h + 1) * dh]
        kh = kv[:, h * dh:(h + 1) * dh]
        vh = kv[:, D + h * dh:D + (h + 1) * dh]
        s = jnp.dot(qh, kh.T, preferred_element_type=jnp.float32)
        p = jnp.exp(s) * maskf
        denom = jnp.sum(p, axis=1, keepdims=True)
        pb = p.astype(jnp.bfloat16)
        ctxs.append(jnp.dot(pb, vh, preferred_element_type=jnp.float32) / denom)
    ctx = jnp.concatenate(ctxs, axis=1).astype(jnp.bfloat16)
    attn_out = jnp.dot(ctx, wo_ref[...], preferred_element_type=jnp.float32)
    r = x_ref[...] + attn_out
    mu = jnp.mean(r, axis=1, keepdims=True)
    var = jnp.mean((r - mu) * (r - mu), axis=1, keepdims=True)
    hh = (r - mu) * jax.lax.rsqrt(var + 1e-5) * g1_ref[...] + b1n_ref[...]
    a = jnp.dot(hh.astype(jnp.bfloat16), w1_ref[...],
                preferred_element_type=jnp.float32) + bf1_ref[...]
    ge = jax.nn.gelu(a)
    o = jnp.dot(ge.astype(jnp.bfloat16), w2_ref[...],
                preferred_element_type=jnp.float32) + bf2_ref[...]
    r2 = hh + o
    mu2 = jnp.mean(r2, axis=1, keepdims=True)
    var2 = jnp.mean((r2 - mu2) * (r2 - mu2), axis=1, keepdims=True)
    o_ref[...] = (r2 - mu2) * jax.lax.rsqrt(var2 + 1e-5) * g2_ref[...] + b2n_ref[...]


def kernel(x, proj, Wq, Wk, Wv, Wo, ln1_g, ln1_b, ln2_g, ln2_b, W1, b1, W2, b2):
    B, S, D = x.shape
    F = W1.shape[1]
    xs = x.reshape(S, D)
    Wqkv = jnp.concatenate([Wq, Wk, Wv], axis=1)
    BR1 = 512
    BR = 256
    nblk = S // BR

    mask, q, kv = pl.pallas_call(
        _dist_qkv_kernel,
        grid=(S // BR1,),
        in_specs=[
            pl.BlockSpec((BR1, D), lambda i: (i, 0)),
            pl.BlockSpec((S, D), lambda i: (0, 0)),
            pl.BlockSpec((D, 3 * D), lambda i: (0, 0)),
        ],
        out_specs=[
            pl.BlockSpec((BR1, S), lambda i: (i, 0)),
            pl.BlockSpec((BR1, D), lambda i: (i, 0)),
            pl.BlockSpec((BR1, 2 * D), lambda i: (i, 0)),
        ],
        out_shape=[
            jax.ShapeDtypeStruct((S, S), jnp.int8),
            jax.ShapeDtypeStruct((S, D), jnp.bfloat16),
            jax.ShapeDtypeStruct((S, 2 * D), jnp.bfloat16),
        ],
    )(xs, xs, Wqkv)

    out = pl.pallas_call(
        _attn_ffn_kernel,
        grid=(nblk,),
        in_specs=[
            pl.BlockSpec((BR, D), lambda i: (i, 0)),
            pl.BlockSpec((S, 2 * D), lambda i: (0, 0)),
            pl.BlockSpec((BR, S), lambda i: (i, 0)),
            pl.BlockSpec((BR, D), lambda i: (i, 0)),
            pl.BlockSpec((D, D), lambda i: (0, 0)),
            pl.BlockSpec((1, D), lambda i: (0, 0)),
            pl.BlockSpec((1, D), lambda i: (0, 0)),
            pl.BlockSpec((D, F), lambda i: (0, 0)),
            pl.BlockSpec((1, F), lambda i: (0, 0)),
            pl.BlockSpec((F, D), lambda i: (0, 0)),
            pl.BlockSpec((1, D), lambda i: (0, 0)),
            pl.BlockSpec((1, D), lambda i: (0, 0)),
            pl.BlockSpec((1, D), lambda i: (0, 0)),
        ],
        out_specs=pl.BlockSpec((BR, D), lambda i: (i, 0)),
        out_shape=jax.ShapeDtypeStruct((S, D), jnp.float32),
    )(q, kv, mask, xs, Wo.astype(jnp.bfloat16),
      ln1_g.reshape(1, D), ln1_b.reshape(1, D),
      W1.astype(jnp.bfloat16), b1.reshape(1, F),
      W2.astype(jnp.bfloat16), b2.reshape(1, D),
      ln2_g.reshape(1, D), ln2_b.reshape(1, D))

    return out.reshape(B, S, D)


# speedup vs baseline: 1.0871x; 1.0871x over previous
import jax
import jax.numpy as jnp
import numpy as np
from jax.experimental import pallas as pl

K_NBR = 32
HEADS = 12


def _dist_qkv_kernel(xb_ref, xf_ref, wq_ref, wk_ref, wv_ref, mask_ref,
                     q_ref, kv_ref):
    blk = pl.program_id(0)
    xb = xb_ref[...]
    xf = xf_ref[...]
    BR, D = xb.shape
    S = xf.shape[0]
    g = jnp.dot(xb, xf.T, preferred_element_type=jnp.float32)
    x2b = jnp.sum(xb * xb, axis=1)
    x2f = jnp.sum(xf * xf, axis=1)
    d2 = x2b[:, None] + x2f[None, :] - 2.0 * g
    d2 = jnp.maximum(d2, 0.0)
    rows = jax.lax.broadcasted_iota(jnp.int32, (BR, S), 0) + blk * BR
    cols = jax.lax.broadcasted_iota(jnp.int32, (BR, S), 1)
    d2 = jnp.where(rows == cols, jnp.inf, d2)
    d = d2
    m = None
    for _ in range(K_NBR):
        m = jnp.min(d, axis=1, keepdims=True)
        d = jnp.where(d <= m, jnp.inf, d)
    mask_ref[...] = (d2 <= m).astype(jnp.int8)
    scale = 1.0 / np.sqrt(D // HEADS)
    q_ref[...] = jnp.dot(xb, wq_ref[...],
                         preferred_element_type=jnp.float32) * scale
    kv_ref[:, :D] = jnp.dot(xb, wk_ref[...], preferred_element_type=jnp.float32)
    kv_ref[:, D:] = jnp.dot(xb, wv_ref[...], preferred_element_type=jnp.float32)


def _attn_ffn_kernel(q_ref, kv_ref, mask_ref, x_ref, wo_ref, g1_ref,
                     b1n_ref, w1_ref, bf1_ref, w2_ref, bf2_ref, g2_ref,
                     b2n_ref, o_ref):
    D = q_ref.shape[1]
    dh = D // HEADS
    q = q_ref[...]
    kv = kv_ref[...]
    maskf = mask_ref[...].astype(jnp.float32)
    ctxs = []
    for h in range(HEADS):
        qh = q[:, h * dh:(h + 1) * dh]
        kh = kv[:, h * dh:(h + 1) * dh]
        vh = kv[:, D + h * dh:D + (h + 1) * dh]
        s = jnp.dot(qh, kh.T, preferred_element_type=jnp.float32)
        p = jnp.exp(s) * maskf
        denom = jnp.sum(p, axis=1, keepdims=True)
        ctxs.append(jnp.dot(p, vh, preferred_element_type=jnp.float32) / denom)
    ctx = jnp.concatenate(ctxs, axis=1)
    attn_out = jnp.dot(ctx, wo_ref[...], preferred_element_type=jnp.float32)
    r = x_ref[...] + attn_out
    mu = jnp.mean(r, axis=1, keepdims=True)
    var = jnp.mean((r - mu) * (r - mu), axis=1, keepdims=True)
    hh = (r - mu) * jax.lax.rsqrt(var + 1e-5) * g1_ref[...] + b1n_ref[...]
    a = jnp.dot(hh, w1_ref[...], preferred_element_type=jnp.float32) + bf1_ref[...]
    ge = jax.nn.gelu(a)
    o = jnp.dot(ge, w2_ref[...], preferred_element_type=jnp.float32) + bf2_ref[...]
    r2 = hh + o
    mu2 = jnp.mean(r2, axis=1, keepdims=True)
    var2 = jnp.mean((r2 - mu2) * (r2 - mu2), axis=1, keepdims=True)
    o_ref[...] = (r2 - mu2) * jax.lax.rsqrt(var2 + 1e-5) * g2_ref[...] + b2n_ref[...]


def kernel(x, proj, Wq, Wk, Wv, Wo, ln1_g, ln1_b, ln2_g, ln2_b, W1, b1, W2, b2):
    B, S, D = x.shape
    F = W1.shape[1]
    xs = x.reshape(S, D)
    BR1 = 512
    BR = 256
    nblk = S // BR

    mask, q, kv = pl.pallas_call(
        _dist_qkv_kernel,
        grid=(S // BR1,),
        in_specs=[
            pl.BlockSpec((BR1, D), lambda i: (i, 0)),
            pl.BlockSpec((S, D), lambda i: (0, 0)),
            pl.BlockSpec((D, D), lambda i: (0, 0)),
            pl.BlockSpec((D, D), lambda i: (0, 0)),
            pl.BlockSpec((D, D), lambda i: (0, 0)),
        ],
        out_specs=[
            pl.BlockSpec((BR1, S), lambda i: (i, 0)),
            pl.BlockSpec((BR1, D), lambda i: (i, 0)),
            pl.BlockSpec((BR1, 2 * D), lambda i: (i, 0)),
        ],
        out_shape=[
            jax.ShapeDtypeStruct((S, S), jnp.int8),
            jax.ShapeDtypeStruct((S, D), jnp.float32),
            jax.ShapeDtypeStruct((S, 2 * D), jnp.float32),
        ],
    )(xs, xs, Wq, Wk, Wv)

    out = pl.pallas_call(
        _attn_ffn_kernel,
        grid=(nblk,),
        in_specs=[
            pl.BlockSpec((BR, D), lambda i: (i, 0)),
            pl.BlockSpec((S, 2 * D), lambda i: (0, 0)),
            pl.BlockSpec((BR, S), lambda i: (i, 0)),
            pl.BlockSpec((BR, D), lambda i: (i, 0)),
            pl.BlockSpec((D, D), lambda i: (0, 0)),
            pl.BlockSpec((1, D), lambda i: (0, 0)),
            pl.BlockSpec((1, D), lambda i: (0, 0)),
            pl.BlockSpec((D, F), lambda i: (0, 0)),
            pl.BlockSpec((1, F), lambda i: (0, 0)),
            pl.BlockSpec((F, D), lambda i: (0, 0)),
            pl.BlockSpec((1, D), lambda i: (0, 0)),
            pl.BlockSpec((1, D), lambda i: (0, 0)),
            pl.BlockSpec((1, D), lambda i: (0, 0)),
        ],
        out_specs=pl.BlockSpec((BR, D), lambda i: (i, 0)),
        out_shape=jax.ShapeDtypeStruct((S, D), jnp.float32),
    )(q, kv, mask, xs, Wo, ln1_g.reshape(1, D), ln1_b.reshape(1, D),
      W1, b1.reshape(1, F), W2, b2.reshape(1, D),
      ln2_g.reshape(1, D), ln2_b.reshape(1, D))

    return out.reshape(B, S, D)


# speedup vs baseline: 1.1041x; 1.0156x over previous
import jax
import jax.numpy as jnp
import numpy as np
from jax.experimental import pallas as pl
from jax.experimental.pallas import tpu as pltpu

K_NBR = 32
HEADS = 12


def _fused_kernel(xb_ref, xf_ref, wq_ref, wk_ref, wv_ref, wo_ref, g1_ref,
                  b1n_ref, w1_ref, bf1_ref, w2_ref, bf2_ref, g2_ref,
                  b2n_ref, o_ref, mask_s, q_s, kv_s):
    phase = pl.program_id(0)
    i = pl.program_id(1)
    BR, D = xb_ref.shape
    dh = D // HEADS

    @pl.when(phase == 0)
    def _dist_qkv():
        xb = xb_ref[...]
        xf = xf_ref[...]
        S = xf.shape[0]
        g = jnp.dot(xb, xf.T, preferred_element_type=jnp.float32)
        x2b = jnp.sum(xb * xb, axis=1)
        x2f = jnp.sum(xf * xf, axis=1)
        d2 = x2b[:, None] + x2f[None, :] - 2.0 * g
        d2 = jnp.maximum(d2, 0.0)
        rows = jax.lax.broadcasted_iota(jnp.int32, (BR, S), 0) + i * BR
        cols = jax.lax.broadcasted_iota(jnp.int32, (BR, S), 1)
        d2 = jnp.where(rows == cols, jnp.inf, d2)
        d = d2
        m = None
        for _ in range(K_NBR):
            m = jnp.min(d, axis=1, keepdims=True)
            d = jnp.where(d <= m, jnp.inf, d)
        mask_s[pl.ds(i * BR, BR), :] = (d2 <= m).astype(jnp.int8)
        scale = 1.0 / np.sqrt(dh)
        q_s[pl.ds(i * BR, BR), :] = (
            jnp.dot(xb, wq_ref[...], preferred_element_type=jnp.float32)
            * scale).astype(jnp.bfloat16)
        kv_s[pl.ds(i * BR, BR), :D] = jnp.dot(
            xb, wk_ref[...], preferred_element_type=jnp.float32
        ).astype(jnp.bfloat16)
        kv_s[pl.ds(i * BR, BR), D:] = jnp.dot(
            xb, wv_ref[...], preferred_element_type=jnp.float32
        ).astype(jnp.bfloat16)

    @pl.when(phase == 1)
    def _attn_ffn():
        q = q_s[pl.ds(i * BR, BR), :]
        maskf = mask_s[pl.ds(i * BR, BR), :].astype(jnp.float32)
        ctxs = []
        for h in range(HEADS):
            qh = q[:, h * dh:(h + 1) * dh]
            kh = kv_s[:, h * dh:(h + 1) * dh]
            vh = kv_s[:, D + h * dh:D + (h + 1) * dh].astype(jnp.float32)
            s = jnp.dot(qh, kh.T, preferred_element_type=jnp.float32)
            p = jnp.exp(s) * maskf
            denom = jnp.sum(p, axis=1, keepdims=True)
            ctxs.append(
                jnp.dot(p, vh, preferred_element_type=jnp.float32) / denom)
        ctx = jnp.concatenate(ctxs, axis=1)
        attn_out = jnp.dot(ctx, wo_ref[...], preferred_element_type=jnp.float32)
        r = xb_ref[...] + attn_out
        mu = jnp.mean(r, axis=1, keepdims=True)
        var = jnp.mean((r - mu) * (r - mu), axis=1, keepdims=True)
        hh = (r - mu) * jax.lax.rsqrt(var + 1e-5) * g1_ref[...] + b1n_ref[...]
        a = jnp.dot(hh, w1_ref[...],
                    preferred_element_type=jnp.float32) + bf1_ref[...]
        ge = jax.nn.gelu(a)
        o = jnp.dot(ge, w2_ref[...],
                    preferred_element_type=jnp.float32) + bf2_ref[...]
        r2 = hh + o
        mu2 = jnp.mean(r2, axis=1, keepdims=True)
        var2 = jnp.mean((r2 - mu2) * (r2 - mu2), axis=1, keepdims=True)
        o_ref[...] = ((r2 - mu2) * jax.lax.rsqrt(var2 + 1e-5) * g2_ref[...]
                      + b2n_ref[...])


def kernel(x, proj, Wq, Wk, Wv, Wo, ln1_g, ln1_b, ln2_g, ln2_b, W1, b1, W2, b2):
    B, S, D = x.shape
    F = W1.shape[1]
    xs = x.reshape(S, D)
    BR = 256
    nblk = S // BR

    out = pl.pallas_call(
        _fused_kernel,
        grid=(2, nblk),
        in_specs=[
            pl.BlockSpec((BR, D), lambda p, i: (i, 0)),
            pl.BlockSpec((S, D), lambda p, i: (0, 0)),
            pl.BlockSpec((D, D), lambda p, i: (0, 0)),
            pl.BlockSpec((D, D), lambda p, i: (0, 0)),
            pl.BlockSpec((D, D), lambda p, i: (0, 0)),
            pl.BlockSpec((D, D), lambda p, i: (0, 0)),
            pl.BlockSpec((1, D), lambda p, i: (0, 0)),
            pl.BlockSpec((1, D), lambda p, i: (0, 0)),
            pl.BlockSpec((D, F), lambda p, i: (0, 0)),
            pl.BlockSpec((1, F), lambda p, i: (0, 0)),
            pl.BlockSpec((F, D), lambda p, i: (0, 0)),
            pl.BlockSpec((1, D), lambda p, i: (0, 0)),
            pl.BlockSpec((1, D), lambda p, i: (0, 0)),
            pl.BlockSpec((1, D), lambda p, i: (0, 0)),
        ],
        out_specs=pl.BlockSpec((BR, D), lambda p, i: (p * i, 0)),
        out_shape=jax.ShapeDtypeStruct((S, D), jnp.float32),
        scratch_shapes=[
            pltpu.VMEM((S, S), jnp.int8),
            pltpu.VMEM((S, D), jnp.bfloat16),
            pltpu.VMEM((S, 2 * D), jnp.bfloat16),
        ],
    )(xs, xs, Wq, Wk, Wv, Wo, ln1_g.reshape(1, D), ln1_b.reshape(1, D),
      W1, b1.reshape(1, F), W2, b2.reshape(1, D),
      ln2_g.reshape(1, D), ln2_b.reshape(1, D))

    return out.reshape(B, S, D)
